# Initial kernel scaffold; baseline (speedup 1.0000x reference)
import jax
import jax.numpy as jnp
from jax.experimental import pallas as pl


def kernel(batch_s, batch_r, batch_o, labels, ent_re, ent_im, ent, rel_re, rel_im, rel):
    raise NotImplementedError("write your pallas kernel here")


# speedup vs baseline: 1.3828x; 1.3828x over previous
import functools

import jax
import jax.numpy as jnp
from jax import lax
from jax.experimental import pallas as pl
from jax.experimental.pallas import tpu as pltpu
from jax.experimental.pallas import tpu_sc as plsc

NUM_ENTS = 100000
NUM_RELS = 1000
HIDDEN = 128
HALF = HIDDEN // 2
BATCH = 16384
LMBDA = 0.01

NC = 2
NS = 16
NW = NC * NS
B_PER_W = BATCH // NW
CHUNK = 128
N_CHUNKS = B_PER_W // CHUNK
L = 16


def _sc_kernel(bs, br, bo, ent_re, ent_im, ent, rel_re, rel_im, rel,
               res_out, sq_out,
               idx_s, idx_r, idx_o,
               g_res, g_ims, g_es, g_rre, g_rim, g_r, g_reo, g_imo, g_eo,
               res_v, sq_v, sem):
    wid = lax.axis_index("s") * NC + lax.axis_index("c")

    accs = tuple(jnp.zeros((L,), jnp.float32) for _ in range(9))

    for ci in range(N_CHUNKS):
        base = wid * B_PER_W + ci * CHUNK
        pltpu.sync_copy(bs.at[pl.ds(base, CHUNK)], idx_s)
        pltpu.sync_copy(br.at[pl.ds(base, CHUNK)], idx_r)
        pltpu.sync_copy(bo.at[pl.ds(base, CHUNK)], idx_o)
        handles = [
            pltpu.async_copy(ent_re.at[idx_s], g_res, sem),
            pltpu.async_copy(ent_im.at[idx_s], g_ims, sem),
            pltpu.async_copy(ent.at[idx_s], g_es, sem),
            pltpu.async_copy(rel_re.at[idx_r], g_rre, sem),
            pltpu.async_copy(rel_im.at[idx_r], g_rim, sem),
            pltpu.async_copy(rel.at[idx_r], g_r, sem),
            pltpu.async_copy(ent_re.at[idx_o], g_reo, sem),
            pltpu.async_copy(ent_im.at[idx_o], g_imo, sem),
            pltpu.async_copy(ent.at[idx_o], g_eo, sem),
        ]
        for h in handles:
            h.wait()

        def body(i, carry):
            a0, a1, a2, a3, a4, a5, a6, a7, a8 = carry
            racc = jnp.zeros((L,), jnp.float32)
            for j in range(HALF // L):
                sl = pl.ds(j * L, L)
                a = g_rre[i, sl]
                b = g_rim[i, sl]
                c = g_res[i, sl]
                d = g_ims[i, sl]
                e = g_reo[i, sl]
                f = g_imo[i, sl]
                t1 = a * c - b * d
                t2 = a * d + b * c
                racc = racc + t1 * e + t2 * f
                a0 = a0 + c * c
                a1 = a1 + d * d
                a3 = a3 + e * e
                a4 = a4 + f * f
                a6 = a6 + a * a
                a7 = a7 + b * b
            for j in range(HIDDEN // L):
                sl = pl.ds(j * L, L)
                g = g_r[i, sl]
                h = g_es[i, sl]
                o = g_eo[i, sl]
                racc = racc + g * h * o
                a2 = a2 + h * h
                a5 = a5 + o * o
                a8 = a8 + g * g
            res_v[i, :] = racc
            return (a0, a1, a2, a3, a4, a5, a6, a7, a8)

        accs = lax.fori_loop(0, CHUNK, body, accs)
        pltpu.sync_copy(res_v, res_out.at[pl.ds(base, CHUNK)])

    for k in range(9):
        sq_v[k, :] = accs[k]
    pltpu.sync_copy(sq_v, sq_out.at[wid])


_sc_call = functools.partial(
    pl.kernel,
    out_type=[
        jax.ShapeDtypeStruct((BATCH, L), jnp.float32),
        jax.ShapeDtypeStruct((NW, 9, L), jnp.float32),
    ],
    mesh=plsc.VectorSubcoreMesh(core_axis_name="c", subcore_axis_name="s"),
    compiler_params=pltpu.CompilerParams(use_tc_tiling_on_sc=False),
    scratch_types=[
        pltpu.VMEM((CHUNK,), jnp.int32),
        pltpu.VMEM((CHUNK,), jnp.int32),
        pltpu.VMEM((CHUNK,), jnp.int32),
        pltpu.VMEM((CHUNK, HALF), jnp.float32),
        pltpu.VMEM((CHUNK, HALF), jnp.float32),
        pltpu.VMEM((CHUNK, HIDDEN), jnp.float32),
        pltpu.VMEM((CHUNK, HALF), jnp.float32),
        pltpu.VMEM((CHUNK, HALF), jnp.float32),
        pltpu.VMEM((CHUNK, HIDDEN), jnp.float32),
        pltpu.VMEM((CHUNK, HALF), jnp.float32),
        pltpu.VMEM((CHUNK, HALF), jnp.float32),
        pltpu.VMEM((CHUNK, HIDDEN), jnp.float32),
        pltpu.VMEM((CHUNK, L), jnp.float32),
        pltpu.VMEM((9, L), jnp.float32),
        pltpu.SemaphoreType.DMA,
    ],
)(_sc_kernel)


def _tc_finish(res_ref, lab_ref, sq_ref, out_ref):
    res = jnp.sum(res_ref[...], axis=1, keepdims=True)
    lab = lab_ref[...]
    sp = jnp.logaddexp(0.0, -lab * res)
    emb_loss = jnp.sum(sp) / BATCH

    c_half = float(BATCH * HALF)
    c_full = float(BATCH * HIDDEN)
    s = [jnp.sum(sq_ref[k:k + 1, :]) for k in range(9)]
    regul = (s[0] / c_half
             + (s[1] / c_half) * (s[2] / c_full)
             + s[3] / c_half
             + s[4] / c_half
             + s[5] / c_full
             + s[6] / c_half
             + s[7] / c_half
             + s[8] / c_full)
    out_ref[...] = jnp.broadcast_to(emb_loss + LMBDA * regul, (1, 1))


def kernel(batch_s, batch_r, batch_o, labels, ent_re, ent_im, ent,
           rel_re, rel_im, rel):
    bs = batch_s.astype(jnp.int32)
    br = batch_r.astype(jnp.int32)
    bo = batch_o.astype(jnp.int32)

    res, sq = _sc_call(bs, br, bo, ent_re, ent_im, ent, rel_re, rel_im, rel)

    lab2d = labels.astype(jnp.float32).reshape(BATCH, 1)
    sq2d = jnp.transpose(sq, (1, 0, 2)).reshape(9, NW * L)

    loss2d = pl.pallas_call(
        _tc_finish,
        out_shape=jax.ShapeDtypeStruct((1, 1), jnp.float32),
    )(res, lab2d, sq2d)
    return loss2d[0, 0]
